# baseline (speedup 1.0000x reference)
import functools

import jax
import jax.numpy as jnp
from jax import lax
from jax.experimental import pallas as pl
from jax.experimental.pallas import tpu as pltpu
from jax.experimental.pallas import tpu_sc as plsc

_N_TREES = 100
_NUM_LEAVES = 1024
_EMB = 16
_BATCH = 16384

_NC = 2
_NS = 16
_NW = _NC * _NS

_HALF = _BATCH // 2
_CHUNK = 1024
_BCHUNKS = _HALF // _CHUNK
_N_UNITS = _N_TREES * _BCHUNKS
_UNITS_PER_W = _N_UNITS // _NW


def _sc_gather(tables, leaves_t):
  mesh = plsc.VectorSubcoreMesh(
      core_axis_name="c", subcore_axis_name="s",
      num_cores=_NC, num_subcores=_NS)

  @functools.partial(
      pl.kernel,
      out_type=jax.ShapeDtypeStruct((_HALF, _N_TREES * _EMB), jnp.float32),
      mesh=mesh,
      scratch_types=[
          pltpu.VMEM_SHARED((_N_TREES // _NC, _NUM_LEAVES, _EMB), jnp.float32),
          pltpu.VMEM((_CHUNK,), jnp.int32),
          pltpu.VMEM((_CHUNK,), jnp.int32),
          pltpu.VMEM((_CHUNK, _EMB), jnp.float32),
          pltpu.VMEM((_CHUNK, _EMB), jnp.float32),
          pltpu.SemaphoreType.DMA,
          pltpu.SemaphoreType.DMA,
          pltpu.SemaphoreType.DMA,
          pltpu.SemaphoreType.DMA,
          pltpu.SemaphoreType.DMA,
          pltpu.SemaphoreType.DMA,
      ],
      compiler_params=pltpu.CompilerParams(use_tc_tiling_on_sc=False),
  )
  def k(tables_hbm, leaves_hbm, out_hbm,
        table_sh, idx0, idx1, rows0, rows1,
        si0, si1, sg0, sg1, sw0, sw1):
    sid = lax.axis_index("s")
    cid = lax.axis_index("c")
    idx = (idx0, idx1)
    rows = (rows0, rows1)
    si = (si0, si1)
    sg = (sg0, sg1)
    sw = (sw0, sw1)
    tpc = _N_TREES // _NC

    pltpu.sync_copy(
        tables_hbm.at[pl.ds(cid * tpc, tpc),
                      pl.ds(sid * (_NUM_LEAVES // _NS), _NUM_LEAVES // _NS), :],
        table_sh.at[:, pl.ds(sid * (_NUM_LEAVES // _NS), _NUM_LEAVES // _NS), :])
    plsc.subcore_barrier()

    def unit(kk):
      u = sid + _NS * kk
      return u // _BCHUNKS, (u % _BCHUNKS) * _CHUNK

    t0, b0 = unit(0)
    pltpu.async_copy(
        leaves_hbm.at[cid * tpc + t0, pl.ds(b0, _CHUNK)], idx[0], si[0])

    for kk in range(_UNITS_PER_W):
      b = kk & 1
      t, bb = unit(kk)
      tg = cid * tpc + t
      pltpu.make_async_copy(
          leaves_hbm.at[tg, pl.ds(bb, _CHUNK)], idx[b], si[b]).wait()
      if kk >= 2:
        tp, bp = unit(kk - 2)
        pltpu.make_async_copy(
            rows[b],
            out_hbm.at[pl.ds(bp, _CHUNK),
                       pl.ds((cid * tpc + tp) * _EMB, _EMB)],
            sw[b]).wait()
      pltpu.async_copy(table_sh.at[t].at[idx[b]], rows[b], sg[b])
      if kk + 1 < _UNITS_PER_W:
        tn, bn = unit(kk + 1)
        pltpu.async_copy(
            leaves_hbm.at[cid * tpc + tn, pl.ds(bn, _CHUNK)],
            idx[1 - b], si[1 - b])
      pltpu.make_async_copy(
          table_sh.at[t].at[idx[b]], rows[b], sg[b]).wait()
      pltpu.async_copy(
          rows[b],
          out_hbm.at[pl.ds(bb, _CHUNK), pl.ds(tg * _EMB, _EMB)],
          sw[b])

    for kk in (_UNITS_PER_W - 2, _UNITS_PER_W - 1):
      b = kk & 1
      t, bb = unit(kk)
      pltpu.make_async_copy(
          rows[b],
          out_hbm.at[pl.ds(bb, _CHUNK),
                     pl.ds((cid * tpc + t) * _EMB, _EMB)],
          sw[b]).wait()

  return k(tables, leaves_t)


@jax.jit
def kernel(leaves, tables):
  leaves_t = leaves.T
  h0 = _sc_gather(tables, leaves_t[:, :_HALF])
  h1 = _sc_gather(tables, leaves_t[:, _HALF:])
  return jnp.concatenate([h0, h1], axis=0)


# speedup vs baseline: 1.1972x; 1.1972x over previous
import functools

import jax
import jax.numpy as jnp
from jax import lax
from jax.experimental import pallas as pl
from jax.experimental.pallas import tpu as pltpu
from jax.experimental.pallas import tpu_sc as plsc

_N_TREES = 100
_NUM_LEAVES = 1024
_EMB = 16
_BATCH = 16384

_NC = 2
_NS = 16
_NW = _NC * _NS

_CHUNK = 2048
_BCHUNKS = _BATCH // _CHUNK
_N_UNITS = _N_TREES * _BCHUNKS
_UNITS_PER_W = _N_UNITS // _NW


def _sc_gather(tables, leaves_t):
  mesh = plsc.VectorSubcoreMesh(
      core_axis_name="c", subcore_axis_name="s",
      num_cores=_NC, num_subcores=_NS)

  @functools.partial(
      pl.kernel,
      out_type=jax.ShapeDtypeStruct((_BATCH, _N_TREES * _EMB), jnp.float32),
      mesh=mesh,
      scratch_types=[
          pltpu.VMEM_SHARED((_N_TREES // _NC, _NUM_LEAVES, _EMB), jnp.float32),
          pltpu.VMEM((_CHUNK,), jnp.int32),
          pltpu.VMEM((_CHUNK,), jnp.int32),
          pltpu.VMEM((_CHUNK, _EMB), jnp.float32),
          pltpu.VMEM((_CHUNK, _EMB), jnp.float32),
          pltpu.SemaphoreType.DMA,
          pltpu.SemaphoreType.DMA,
          pltpu.SemaphoreType.DMA,
          pltpu.SemaphoreType.DMA,
          pltpu.SemaphoreType.DMA,
          pltpu.SemaphoreType.DMA,
      ],
      compiler_params=pltpu.CompilerParams(use_tc_tiling_on_sc=False),
  )
  def k(tables_hbm, leaves_hbm, out_hbm,
        table_sh, idx0, idx1, rows0, rows1,
        si0, si1, sg0, sg1, sw0, sw1):
    sid = lax.axis_index("s")
    cid = lax.axis_index("c")
    idx = (idx0, idx1)
    rows = (rows0, rows1)
    si = (si0, si1)
    sg = (sg0, sg1)
    sw = (sw0, sw1)
    tpc = _N_TREES // _NC

    pltpu.sync_copy(
        tables_hbm.at[pl.ds(cid * tpc, tpc),
                      pl.ds(sid * (_NUM_LEAVES // _NS), _NUM_LEAVES // _NS), :],
        table_sh.at[:, pl.ds(sid * (_NUM_LEAVES // _NS), _NUM_LEAVES // _NS), :])
    plsc.subcore_barrier()

    def unit(kk):
      u = sid + _NS * kk
      return u // _BCHUNKS, (u % _BCHUNKS) * _CHUNK

    t0, b0 = unit(0)
    pltpu.async_copy(
        leaves_hbm.at[cid * tpc + t0, pl.ds(b0, _CHUNK)], idx[0], si[0])

    for kk in range(_UNITS_PER_W):
      b = kk & 1
      t, bb = unit(kk)
      tg = cid * tpc + t
      pltpu.make_async_copy(
          leaves_hbm.at[tg, pl.ds(bb, _CHUNK)], idx[b], si[b]).wait()
      if kk >= 2:
        tp, bp = unit(kk - 2)
        pltpu.make_async_copy(
            rows[b],
            out_hbm.at[pl.ds(bp, _CHUNK),
                       pl.ds((cid * tpc + tp) * _EMB, _EMB)],
            sw[b]).wait()
      pltpu.async_copy(table_sh.at[t].at[idx[b]], rows[b], sg[b])
      if kk + 1 < _UNITS_PER_W:
        tn, bn = unit(kk + 1)
        pltpu.async_copy(
            leaves_hbm.at[cid * tpc + tn, pl.ds(bn, _CHUNK)],
            idx[1 - b], si[1 - b])
      pltpu.make_async_copy(
          table_sh.at[t].at[idx[b]], rows[b], sg[b]).wait()
      pltpu.async_copy(
          rows[b],
          out_hbm.at[pl.ds(bb, _CHUNK), pl.ds(tg * _EMB, _EMB)],
          sw[b])

    for kk in (_UNITS_PER_W - 2, _UNITS_PER_W - 1):
      b = kk & 1
      t, bb = unit(kk)
      pltpu.make_async_copy(
          rows[b],
          out_hbm.at[pl.ds(bb, _CHUNK),
                     pl.ds((cid * tpc + t) * _EMB, _EMB)],
          sw[b]).wait()

  return k(tables, leaves_t)


@jax.jit
def kernel(leaves, tables):
  leaves_t = leaves.T
  return _sc_gather(tables, leaves_t)
